# baseline (speedup 1.0000x reference)
import jax
import jax.numpy as jnp
from jax.experimental import pallas as pl

BATCH = 4096
INPUT_SIZE = 1024
HIDDEN_SIZE = 2048
NUM_CLASSES = 1000
WTA_K = HIDDEN_SIZE // 2

BLOCK_B = 256


def _body(x_ref, w_ref, fc_ref, o_ref):
    int_min32 = jnp.int32(-(2**31))
    mask31 = jnp.int32(0x7FFFFFFF)
    fr = jnp.dot(x_ref[...], w_ref[...], preferred_element_type=jnp.float32)
    v = jax.lax.bitcast_convert_type(fr, jnp.int32)
    v = jnp.where(v < 0, v ^ mask31, v)

    kf = jnp.float32(WTA_K)
    cnt = jnp.sum(v >= 0, axis=1, keepdims=True, dtype=jnp.float32)
    prefix = jnp.where(cnt >= kf, jnp.int32(0), int_min32)

    def step(i, prefix):
        bit = jnp.int32(1) << (jnp.int32(30) - i)
        cand = prefix | bit
        c = jnp.sum(v >= cand, axis=1, keepdims=True, dtype=jnp.float32)
        return jnp.where(c >= kf, cand, prefix)

    prefix = jax.lax.fori_loop(0, 31, step, prefix, unroll=True)

    mask = (v >= prefix).astype(jnp.float32)
    o_ref[...] = jax.lax.dot_general(
        mask, fc_ref[...], (((1,), (1,)), ((), ())),
        preferred_element_type=jnp.float32)


@jax.jit
def _run(x, fna_weight, fc_weight):
    nb = x.shape[0] // BLOCK_B
    return pl.pallas_call(
        _body,
        grid=(nb,),
        in_specs=[
            pl.BlockSpec((BLOCK_B, INPUT_SIZE), lambda i: (i, 0)),
            pl.BlockSpec((INPUT_SIZE, HIDDEN_SIZE), lambda i: (0, 0)),
            pl.BlockSpec((NUM_CLASSES, HIDDEN_SIZE), lambda i: (0, 0)),
        ],
        out_specs=pl.BlockSpec((BLOCK_B, NUM_CLASSES), lambda i: (i, 0)),
        out_shape=jax.ShapeDtypeStruct((x.shape[0], NUM_CLASSES), jnp.float32),
    )(x, fna_weight, fc_weight)


def kernel(x, fna_weight, fc_weight):
    return _run(x, fna_weight, fc_weight)


# speedup vs baseline: 1.0200x; 1.0200x over previous
import jax
import jax.numpy as jnp
from jax.experimental import pallas as pl

BATCH = 4096
INPUT_SIZE = 1024
HIDDEN_SIZE = 2048
NUM_CLASSES = 1000
WTA_K = HIDDEN_SIZE // 2

BLOCK_B = 512


def _body(x_ref, w_ref, fc_ref, o_ref):
    int_min32 = jnp.int32(-(2**31))
    mask31 = jnp.int32(0x7FFFFFFF)
    fr = jnp.dot(x_ref[...], w_ref[...], preferred_element_type=jnp.float32)
    v = jax.lax.bitcast_convert_type(fr, jnp.int32)
    v = jnp.where(v < 0, v ^ mask31, v)

    kf = jnp.float32(WTA_K)
    cnt = jnp.sum(v >= 0, axis=1, keepdims=True, dtype=jnp.float32)
    prefix = jnp.where(cnt >= kf, jnp.int32(0), int_min32)

    def step(i, prefix):
        bit = jnp.int32(1) << (jnp.int32(30) - i)
        cand = prefix | bit
        c = jnp.sum(v >= cand, axis=1, keepdims=True, dtype=jnp.float32)
        return jnp.where(c >= kf, cand, prefix)

    prefix = jax.lax.fori_loop(0, 31, step, prefix, unroll=True)

    mask = (v >= prefix).astype(jnp.float32)
    o_ref[...] = jax.lax.dot_general(
        mask, fc_ref[...], (((1,), (1,)), ((), ())),
        preferred_element_type=jnp.float32)


@jax.jit
def _run(x, fna_weight, fc_weight):
    nb = x.shape[0] // BLOCK_B
    return pl.pallas_call(
        _body,
        grid=(nb,),
        in_specs=[
            pl.BlockSpec((BLOCK_B, INPUT_SIZE), lambda i: (i, 0)),
            pl.BlockSpec((INPUT_SIZE, HIDDEN_SIZE), lambda i: (0, 0)),
            pl.BlockSpec((NUM_CLASSES, HIDDEN_SIZE), lambda i: (0, 0)),
        ],
        out_specs=pl.BlockSpec((BLOCK_B, NUM_CLASSES), lambda i: (i, 0)),
        out_shape=jax.ShapeDtypeStruct((x.shape[0], NUM_CLASSES), jnp.float32),
    )(x, fna_weight, fc_weight)


def kernel(x, fna_weight, fc_weight):
    return _run(x, fna_weight, fc_weight)
